# baseline (speedup 1.0000x reference)
import jax
import jax.numpy as jnp
from jax import lax
from jax.experimental import pallas as pl
from jax.experimental.pallas import tpu as pltpu
from jax.experimental.pallas import tpu_sc as plsc

N_FIELDS = 26
FIELD_SIZE = 100000
D = 32
DP = 128
B = 16384
V = 2600000
L = 16
NC, NS = 2, 16
NW = NC * NS
BPW = B // NW
IPW = BPW * N_FIELDS
GROWS = 104
GB = GROWS // N_FIELDS
NG = IPW // GROWS
R = 4
NITER = NG // R
PERIOD = 208
INV_N = float(1.0 / N_FIELDS)

DT_W = 512
DT_GRID = -(-V // DT_W)
VP = DT_GRID * DT_W


def _detile_body(in_ref, out_ref, scratch, sem):
    i = pl.program_id(0)
    p = i % 2

    @pl.when(i >= 2)
    def _():
        pltpu.make_async_copy(
            scratch.at[p],
            out_ref.at[pl.ds((i - 2) * DT_W, DT_W)], sem.at[p]).wait()

    scratch[p, :, pl.ds(0, D)] = in_ref[...].T
    pltpu.async_copy(
        scratch.at[p], out_ref.at[pl.ds(i * DT_W, DT_W)], sem.at[p])

    @pl.when(i == DT_GRID - 1)
    def _():
        pltpu.make_async_copy(
            scratch.at[1 - p],
            out_ref.at[pl.ds(0, DT_W)], sem.at[1 - p]).wait()
        pltpu.make_async_copy(
            scratch.at[p],
            out_ref.at[pl.ds(0, DT_W)], sem.at[p]).wait()


def _tc_detile(tt):
    return pl.pallas_call(
        _detile_body,
        grid=(DT_GRID,),
        in_specs=[pl.BlockSpec((D, DT_W), lambda i: (0, i))],
        out_specs=pl.BlockSpec(memory_space=pl.ANY),
        out_shape=jax.ShapeDtypeStruct((VP, DP), jnp.float32),
        scratch_shapes=[
            pltpu.VMEM((2, DT_W, DP), jnp.float32),
            pltpu.SemaphoreType.DMA((2,)),
        ],
    )(tt)


def _fire(table_hbm, idx_v, rows_v, sem, g, b):
    src = table_hbm.at[idx_v.at[pl.ds(g * GROWS, GROWS)]]
    return pltpu.async_copy(src, rows_v.at[b], sem.at[b])


def _sc_body(idx_hbm, table_hbm, out_hbm, idx_v, rows_v, out_v, sem):
    wid = lax.axis_index("s") * NC + lax.axis_index("c")

    pltpu.sync_copy(idx_hbm.at[pl.ds(wid * IPW, IPW)], idx_v)

    def off_body(blk, carry):
        base = blk * PERIOD
        for v in range(PERIOD // L):
            off = ((lax.iota(jnp.int32, L) + v * L) % N_FIELDS) * FIELD_SIZE
            sl = pl.ds(base + v * L, L)
            idx_v[sl] = idx_v[sl] + off
        return carry

    lax.fori_loop(0, IPW // PERIOD, off_body, 0)

    for b in range(R):
        _fire(table_hbm, idx_v, rows_v, sem, b, b)

    def ring_body(i, carry):
        for b in range(R):
            g = i * R + b
            pltpu.make_async_copy(
                table_hbm.at[idx_v.at[pl.ds(0, GROWS)]],
                rows_v.at[b], sem.at[b]).wait()
            for ii in range(GB):
                r0 = ii * N_FIELDS
                acc0 = rows_v[b, r0, pl.ds(0, L)]
                acc1 = rows_v[b, r0, pl.ds(L, L)]
                for j in range(1, N_FIELDS):
                    acc0 = acc0 + rows_v[b, r0 + j, pl.ds(0, L)]
                    acc1 = acc1 + rows_v[b, r0 + j, pl.ds(L, L)]
                orow = g * GB + ii
                out_v[orow, pl.ds(0, L)] = acc0 * INV_N
                out_v[orow, pl.ds(L, L)] = acc1 * INV_N

            @pl.when(i + 1 < NITER)
            def _():
                _fire(table_hbm, idx_v, rows_v, sem, g + R, b)
        return carry

    lax.fori_loop(0, NITER, ring_body, 0)

    pltpu.sync_copy(out_v, out_hbm.at[pl.ds(wid * BPW, BPW)])


@jax.jit
def _sc_embed(idx_flat, table):
    mesh = plsc.VectorSubcoreMesh(core_axis_name="c", subcore_axis_name="s")
    return pl.kernel(
        _sc_body,
        out_type=jax.ShapeDtypeStruct((B, D), jnp.float32),
        mesh=mesh,
        scratch_types=[
            pltpu.VMEM((IPW,), jnp.int32),
            pltpu.VMEM((R, GROWS, DP), jnp.float32),
            pltpu.VMEM((BPW, D), jnp.float32),
            pltpu.SemaphoreType.DMA((R,)),
        ],
        compiler_params=pltpu.CompilerParams(use_tc_tiling_on_sc=False),
    )(idx_flat, table)


def kernel(inputs, embedding):
    table_pad = _tc_detile(embedding.T)
    return _sc_embed(inputs.reshape(-1), table_pad)


# speedup vs baseline: 2.1304x; 2.1304x over previous
import jax
import jax.numpy as jnp
from jax import lax
from jax.experimental import pallas as pl
from jax.experimental.pallas import tpu as pltpu
from jax.experimental.pallas import tpu_sc as plsc

N_FIELDS = 26
FIELD_SIZE = 100000
D = 32
DP = 128
B = 16384
V = 2600000
L = 16
NC, NS = 2, 16
NW = NC * NS
BPW = B // NW
IPW = BPW * N_FIELDS
GROWS = 104
GB = GROWS // N_FIELDS
NG = IPW // GROWS
R = 4
NITER = NG // R
PERIOD = 208
INV_N = float(1.0 / N_FIELDS)

DT_W = 4096
DT_GRID = -(-V // DT_W)
VP = DT_GRID * DT_W


def _detile_body(in_ref, out_ref, scratch, sem):
    i = pl.program_id(0)
    p = i % 2

    @pl.when(i >= 2)
    def _():
        pltpu.make_async_copy(
            scratch.at[p],
            out_ref.at[pl.ds((i - 2) * DT_W, DT_W)], sem.at[p]).wait()

    eye = jnp.eye(D, dtype=jnp.float32)
    xt = lax.dot_general(in_ref[...], eye, (((0,), (0,)), ((), ())),
                         precision=lax.Precision.HIGHEST)
    scratch[p, :, pl.ds(0, D)] = xt
    pltpu.async_copy(
        scratch.at[p], out_ref.at[pl.ds(i * DT_W, DT_W)], sem.at[p])

    @pl.when(i == DT_GRID - 1)
    def _():
        pltpu.make_async_copy(
            scratch.at[1 - p],
            out_ref.at[pl.ds(0, DT_W)], sem.at[1 - p]).wait()
        pltpu.make_async_copy(
            scratch.at[p],
            out_ref.at[pl.ds(0, DT_W)], sem.at[p]).wait()


def _tc_detile(tt):
    return pl.pallas_call(
        _detile_body,
        grid=(DT_GRID,),
        in_specs=[pl.BlockSpec((D, DT_W), lambda i: (0, i))],
        out_specs=pl.BlockSpec(memory_space=pl.ANY),
        out_shape=jax.ShapeDtypeStruct((VP, DP), jnp.float32),
        scratch_shapes=[
            pltpu.VMEM((2, DT_W, DP), jnp.float32),
            pltpu.SemaphoreType.DMA((2,)),
        ],
    )(tt)


def _fire(table_hbm, idx_v, rows_v, sem, g, b):
    src = table_hbm.at[idx_v.at[pl.ds(g * GROWS, GROWS)]]
    return pltpu.async_copy(src, rows_v.at[b], sem.at[b])


def _sc_body(idx_hbm, table_hbm, out_hbm, idx_v, rows_v, out_v, sem):
    wid = lax.axis_index("s") * NC + lax.axis_index("c")

    pltpu.sync_copy(idx_hbm.at[pl.ds(wid * IPW, IPW)], idx_v)

    def off_body(blk, carry):
        base = blk * PERIOD
        for v in range(PERIOD // L):
            off = ((lax.iota(jnp.int32, L) + v * L) % N_FIELDS) * FIELD_SIZE
            sl = pl.ds(base + v * L, L)
            idx_v[sl] = idx_v[sl] + off
        return carry

    lax.fori_loop(0, IPW // PERIOD, off_body, 0)

    for b in range(R):
        _fire(table_hbm, idx_v, rows_v, sem, b, b)

    def ring_body(i, carry):
        for b in range(R):
            g = i * R + b
            pltpu.make_async_copy(
                table_hbm.at[idx_v.at[pl.ds(0, GROWS)]],
                rows_v.at[b], sem.at[b]).wait()
            for ii in range(GB):
                r0 = ii * N_FIELDS
                acc0 = rows_v[b, r0, pl.ds(0, L)]
                acc1 = rows_v[b, r0, pl.ds(L, L)]
                for j in range(1, N_FIELDS):
                    acc0 = acc0 + rows_v[b, r0 + j, pl.ds(0, L)]
                    acc1 = acc1 + rows_v[b, r0 + j, pl.ds(L, L)]
                orow = g * GB + ii
                out_v[orow, pl.ds(0, L)] = acc0 * INV_N
                out_v[orow, pl.ds(L, L)] = acc1 * INV_N

            @pl.when(i + 1 < NITER)
            def _():
                _fire(table_hbm, idx_v, rows_v, sem, g + R, b)
        return carry

    lax.fori_loop(0, NITER, ring_body, 0)

    pltpu.sync_copy(out_v, out_hbm.at[pl.ds(wid * BPW, BPW)])


@jax.jit
def _sc_embed(idx_flat, table):
    mesh = plsc.VectorSubcoreMesh(core_axis_name="c", subcore_axis_name="s")
    return pl.kernel(
        _sc_body,
        out_type=jax.ShapeDtypeStruct((B, D), jnp.float32),
        mesh=mesh,
        scratch_types=[
            pltpu.VMEM((IPW,), jnp.int32),
            pltpu.VMEM((R, GROWS, DP), jnp.float32),
            pltpu.VMEM((BPW, D), jnp.float32),
            pltpu.SemaphoreType.DMA((R,)),
        ],
        compiler_params=pltpu.CompilerParams(use_tc_tiling_on_sc=False),
    )(idx_flat, table)


def kernel(inputs, embedding):
    table_pad = _tc_detile(embedding.T)
    return _sc_embed(inputs.reshape(-1), table_pad)
